# baseline (speedup 1.0000x reference)
import functools

import jax
import jax.numpy as jnp
from jax import lax
from jax.experimental import pallas as pl
from jax.experimental.pallas import tpu as pltpu
from jax.experimental.pallas import tpu_sc as plsc

_NC = 2
_NS = 16


def _sc_gather(nodes_flat, emb_table):
    bn = nodes_flat.shape[0]
    d = emb_table.shape[1]
    nw = _NC * _NS
    b_per_w = bn // nw
    mesh = plsc.VectorSubcoreMesh(core_axis_name="c", subcore_axis_name="s")

    @functools.partial(
        pl.kernel,
        mesh=mesh,
        out_type=jax.ShapeDtypeStruct((bn, d), jnp.float32),
        scratch_types=[
            pltpu.VMEM((b_per_w,), jnp.int32),
            pltpu.VMEM((b_per_w, d), jnp.float32),
            pltpu.SemaphoreType.DMA,
        ],
        compiler_params=pltpu.CompilerParams(use_tc_tiling_on_sc=False),
    )
    def gather_kernel(idx_hbm, table_hbm, out_hbm, idx_v, rows_v, sem):
        wid = lax.axis_index("s") * _NC + lax.axis_index("c")
        base = wid * b_per_w
        pltpu.sync_copy(idx_hbm.at[pl.ds(base, b_per_w)], idx_v)
        pltpu.async_copy(table_hbm.at[idx_v], rows_v, sem).wait()
        pltpu.sync_copy(rows_v, out_hbm.at[pl.ds(base, b_per_w)])

    return gather_kernel(nodes_flat, emb_table)


_G = 16


def _tc_body(adj_ref, npm_ref, h0_ref, tm_ref, pos_ref, w_ref, gw_ref,
             gb_ref, tt_ref, out_ref):
    g, n, _ = adj_ref.shape
    d = h0_ref.shape[-1]
    l = npm_ref.shape[-1]
    npm = npm_ref[...].reshape(g * n, l)
    h0 = h0_ref[...]
    tm4 = tm_ref[...].reshape(g * n, 4)

    f32 = jnp.float32
    ones_row = jnp.ones((1, d), f32)

    kinds = lax.broadcasted_iota(jnp.int32, (g * n, 4), 1)
    onehot = (tm4 == kinds).astype(f32)

    tw = jnp.dot(tt_ref[...], w_ref[d:2 * d, :], preferred_element_type=f32)
    t_term = jnp.dot(onehot, tw, preferred_element_type=f32)
    mrows = (lax.broadcasted_iota(jnp.int32, (4, d), 0) > 0).astype(f32)
    vmask = jnp.dot(onehot, mrows, preferred_element_type=f32)

    pe_raw = jnp.dot(npm, pos_ref[...], preferred_element_type=f32)
    pn1 = jnp.sum(npm, axis=-1, keepdims=True)
    ipn = jnp.dot(1.0 / (pn1 + 1e-9), ones_row, preferred_element_type=f32)
    pe = pe_raw * ipn * vmask

    h = (jnp.dot(h0, w_ref[0:d, :], preferred_element_type=f32)
         + jnp.dot(pe, w_ref[2 * d:3 * d, :], preferred_element_type=f32)
         + t_term)

    a = (adj_ref[...] > 0).astype(f32)
    deg1 = jnp.sum(a, axis=-1, keepdims=True).reshape(g * n, 1)
    rdeg = jnp.dot(1.0 / (deg1 + 1e-9), ones_row, preferred_element_type=f32)

    gw = gw_ref[...]
    gb = gb_ref[0]
    for _ in range(2):
        agg = lax.dot_general(
            a, h.reshape(g, n, d), (((2,), (1,)), ((0,), (0,))),
            preferred_element_type=f32).reshape(g * n, d) * rdeg
        h = jnp.maximum(
            jnp.dot(agg, gw, preferred_element_type=f32) + gb, 0.0) * vmask

    out_ref[...] = h.reshape(g, n, d)


def _tc_call(adj, npm, h0, tm3, pos, w_pos_type, gcn_W, gcn_b2, type_table,
             interpret=False):
    b, n = adj.shape[:2]
    d = h0.shape[-1]
    l = npm.shape[-1]
    g = _G
    return pl.pallas_call(
        _tc_body,
        grid=(b // g,),
        in_specs=[
            pl.BlockSpec((g, n, n), lambda i: (i, 0, 0)),
            pl.BlockSpec((g, n, l), lambda i: (i, 0, 0)),
            pl.BlockSpec((g * n, d), lambda i: (i, 0)),
            pl.BlockSpec((g, n, 4), lambda i: (i, 0, 0)),
            pl.BlockSpec((l, d), lambda i: (0, 0)),
            pl.BlockSpec((3 * d, d), lambda i: (0, 0)),
            pl.BlockSpec((d, d), lambda i: (0, 0)),
            pl.BlockSpec((1, d), lambda i: (0, 0)),
            pl.BlockSpec((4, d), lambda i: (0, 0)),
        ],
        out_specs=pl.BlockSpec((g, n, d), lambda i: (i, 0, 0)),
        out_shape=jax.ShapeDtypeStruct((b, n, d), jnp.float32),
        interpret=interpret,
    )(adj, npm, h0, tm3, pos, w_pos_type, gcn_W, gcn_b2, type_table)


def kernel(adj, nodes, node_type_mask, node_pos_matrix, emb_table, type_table,
           pos_table, w_pos_type, gcn_W, gcn_b):
    b, n = nodes.shape
    d = emb_table.shape[1]
    l = node_pos_matrix.shape[-1]

    nodes_flat = nodes.reshape(-1).astype(jnp.int32)
    h0 = _sc_gather(nodes_flat, emb_table)

    tm4 = jnp.broadcast_to(
        node_type_mask.astype(jnp.int32)[..., None], (b, n, 4))
    return _tc_call(adj, node_pos_matrix, h0, tm4, pos_table[:l],
                    w_pos_type, gcn_W, gcn_b.reshape(1, d), type_table)


# speedup vs baseline: 1.0120x; 1.0120x over previous
import functools

import jax
import jax.numpy as jnp
from jax import lax
from jax.experimental import pallas as pl
from jax.experimental.pallas import tpu as pltpu
from jax.experimental.pallas import tpu_sc as plsc

_NC = 2
_NS = 16


def _sc_gather(nodes_flat, emb_table):
    bn = nodes_flat.shape[0]
    d = emb_table.shape[1]
    nw = _NC * _NS
    b_per_w = bn // nw
    mesh = plsc.VectorSubcoreMesh(core_axis_name="c", subcore_axis_name="s")

    @functools.partial(
        pl.kernel,
        mesh=mesh,
        out_type=jax.ShapeDtypeStruct((bn, d), jnp.float32),
        scratch_types=[
            pltpu.VMEM((b_per_w,), jnp.int32),
            pltpu.VMEM((b_per_w, d), jnp.float32),
            pltpu.SemaphoreType.DMA,
        ],
        compiler_params=pltpu.CompilerParams(use_tc_tiling_on_sc=False),
    )
    def gather_kernel(idx_hbm, table_hbm, out_hbm, idx_v, rows_v, sem):
        wid = lax.axis_index("s") * _NC + lax.axis_index("c")
        base = wid * b_per_w
        pltpu.sync_copy(idx_hbm.at[pl.ds(base, b_per_w)], idx_v)
        pltpu.async_copy(table_hbm.at[idx_v], rows_v, sem).wait()
        pltpu.sync_copy(rows_v, out_hbm.at[pl.ds(base, b_per_w)])

    return gather_kernel(nodes_flat, emb_table)


_G = 8


def _tc_body(adj_ref, npm_ref, h0_ref, tm_ref, pos_ref, w_ref, gw_ref,
             gb_ref, tt_ref, out_ref):
    g, n, _ = adj_ref.shape
    d = h0_ref.shape[-1]
    l = npm_ref.shape[-1]
    npm = npm_ref[...].reshape(g * n, l)
    h0 = h0_ref[...]
    tm4 = tm_ref[...].reshape(g * n, 4)

    f32 = jnp.float32
    ones_row = jnp.ones((1, d), f32)

    kinds = lax.broadcasted_iota(jnp.int32, (g * n, 4), 1)
    onehot = (tm4 == kinds).astype(f32)

    tw = jnp.dot(tt_ref[...], w_ref[d:2 * d, :], preferred_element_type=f32)
    t_term = jnp.dot(onehot, tw, preferred_element_type=f32)
    mrows = (lax.broadcasted_iota(jnp.int32, (4, d), 0) > 0).astype(f32)
    vmask = jnp.dot(onehot, mrows, preferred_element_type=f32)

    pe_raw = jnp.dot(npm, pos_ref[...], preferred_element_type=f32)
    pn1 = jnp.sum(npm, axis=-1, keepdims=True)
    ipn = jnp.dot(1.0 / (pn1 + 1e-9), ones_row, preferred_element_type=f32)
    pe = pe_raw * ipn * vmask

    h = (jnp.dot(h0, w_ref[0:d, :], preferred_element_type=f32)
         + jnp.dot(pe, w_ref[2 * d:3 * d, :], preferred_element_type=f32)
         + t_term)

    a = (adj_ref[...] > 0).astype(f32)
    deg1 = jnp.sum(a, axis=-1, keepdims=True).reshape(g * n, 1)
    rdeg = jnp.dot(1.0 / (deg1 + 1e-9), ones_row, preferred_element_type=f32)

    gw = gw_ref[...]
    gb = gb_ref[0]
    for _ in range(2):
        agg = lax.dot_general(
            a, h.reshape(g, n, d), (((2,), (1,)), ((0,), (0,))),
            preferred_element_type=f32).reshape(g * n, d) * rdeg
        h = jnp.maximum(
            jnp.dot(agg, gw, preferred_element_type=f32) + gb, 0.0) * vmask

    out_ref[...] = h.reshape(g, n, d)


def _tc_call(adj, npm, h0, tm3, pos, w_pos_type, gcn_W, gcn_b2, type_table,
             interpret=False):
    b, n = adj.shape[:2]
    d = h0.shape[-1]
    l = npm.shape[-1]
    g = _G
    return pl.pallas_call(
        _tc_body,
        grid=(b // g,),
        in_specs=[
            pl.BlockSpec((g, n, n), lambda i: (i, 0, 0)),
            pl.BlockSpec((g, n, l), lambda i: (i, 0, 0)),
            pl.BlockSpec((g * n, d), lambda i: (i, 0)),
            pl.BlockSpec((g, n, 4), lambda i: (i, 0, 0)),
            pl.BlockSpec((l, d), lambda i: (0, 0)),
            pl.BlockSpec((3 * d, d), lambda i: (0, 0)),
            pl.BlockSpec((d, d), lambda i: (0, 0)),
            pl.BlockSpec((1, d), lambda i: (0, 0)),
            pl.BlockSpec((4, d), lambda i: (0, 0)),
        ],
        out_specs=pl.BlockSpec((g, n, d), lambda i: (i, 0, 0)),
        out_shape=jax.ShapeDtypeStruct((b, n, d), jnp.float32),
        compiler_params=pltpu.CompilerParams(
            dimension_semantics=("parallel",)),
        interpret=interpret,
    )(adj, npm, h0, tm3, pos, w_pos_type, gcn_W, gcn_b2, type_table)


def kernel(adj, nodes, node_type_mask, node_pos_matrix, emb_table, type_table,
           pos_table, w_pos_type, gcn_W, gcn_b):
    b, n = nodes.shape
    d = emb_table.shape[1]
    l = node_pos_matrix.shape[-1]

    nodes_flat = nodes.reshape(-1).astype(jnp.int32)
    h0 = _sc_gather(nodes_flat, emb_table)

    tm4 = jnp.broadcast_to(
        node_type_mask.astype(jnp.int32)[..., None], (b, n, 4))
    return _tc_call(adj, node_pos_matrix, h0, tm4, pos_table[:l],
                    w_pos_type, gcn_W, gcn_b.reshape(1, d), type_table)
